# Initial kernel scaffold; baseline (speedup 1.0000x reference)
import jax
import jax.numpy as jnp
from jax.experimental import pallas as pl


def kernel(x, edge_index, temp):
    raise NotImplementedError("write your pallas kernel here")


# speedup vs baseline: 9.6189x; 9.6189x over previous
import functools

import jax
import jax.numpy as jnp
from jax import lax
from jax.experimental import pallas as pl
from jax.experimental.pallas import tpu as pltpu
from jax.experimental.pallas import tpu_sc as plsc

N = 10000
D = 128
E = 320000
K = 10

NC = 2
NS = 16
NW = NC * NS
CHUNK = 128
ET = E + N
CH = -(-ET // (NW * CHUNK))
EP = NW * CH * CHUNK
NPAD = 10112
TR = NPAD // NS
DW = 16
RB = 1000
GRID = N // RB

_mesh = plsc.VectorSubcoreMesh(core_axis_name="c", subcore_axis_name="s")


@functools.partial(
    pl.kernel,
    mesh=_mesh,
    out_type=jax.ShapeDtypeStruct((NC, NPAD, D), jnp.float32),
    scratch_types=[
        pltpu.VMEM((CH, CHUNK), jnp.int32),
        pltpu.VMEM((CH, CHUNK), jnp.int32),
        pltpu.VMEM((CHUNK, D), jnp.float32),
        pltpu.VMEM_SHARED((NPAD, D), jnp.float32),
        pltpu.SemaphoreType.DMA,
    ],
)
def _spmv(g_hbm, rowidx_hbm, colidx_hbm, zeros_hbm, out_hbm,
          row_v, col_v, msg_v, accum, sem):
    cid = lax.axis_index("c")
    sid = lax.axis_index("s")
    wid = sid * NC + cid
    pltpu.sync_copy(rowidx_hbm.at[wid], row_v)
    pltpu.sync_copy(colidx_hbm.at[wid], col_v)
    pltpu.sync_copy(zeros_hbm, accum.at[pl.ds(sid * TR, TR)])
    plsc.subcore_barrier()

    def chunk(j, carry):
        pltpu.async_copy(g_hbm.at[row_v.at[j]], msg_v, sem).wait()
        pltpu.sync_copy(msg_v, accum.at[col_v.at[j]], add=True)
        return carry

    lax.fori_loop(0, CH, chunk, 0)
    plsc.subcore_barrier()
    pltpu.sync_copy(accum.at[pl.ds(sid * TR, TR)],
                    out_hbm.at[cid, pl.ds(sid * TR, TR)])


@functools.partial(
    pl.kernel,
    mesh=_mesh,
    out_type=jax.ShapeDtypeStruct((NC, NPAD, DW), jnp.float32),
    scratch_types=[
        pltpu.VMEM((CH, CHUNK), jnp.int32),
        pltpu.VMEM((CHUNK, DW), jnp.float32),
        pltpu.VMEM_SHARED((NPAD, DW), jnp.float32),
    ],
)
def _deg(colidx_hbm, ones_hbm, zeros_hbm, out_hbm, col_v, ones_v, accum):
    cid = lax.axis_index("c")
    sid = lax.axis_index("s")
    wid = sid * NC + cid
    pltpu.sync_copy(colidx_hbm.at[wid], col_v)
    pltpu.sync_copy(ones_hbm, ones_v)
    pltpu.sync_copy(zeros_hbm, accum.at[pl.ds(sid * TR, TR)])
    plsc.subcore_barrier()

    def chunk(j, carry):
        pltpu.sync_copy(ones_v, accum.at[col_v.at[j]], add=True)
        return carry

    lax.fori_loop(0, CH, chunk, 0)
    plsc.subcore_barrier()
    pltpu.sync_copy(accum.at[pl.ds(sid * TR, TR)],
                    out_hbm.at[cid, pl.ds(sid * TR, TR)])


def _prep_body(t0_ref, x_ref, d0_ref, d1_ref, dis_ref, g_ref, hid_ref):
    deg = d0_ref[0][:, 0:1] + d1_ref[0][:, 0:1]
    dis = lax.rsqrt(deg)
    x = x_ref[...]
    dis_ref[...] = dis
    g_ref[...] = dis * x
    hid_ref[...] = t0_ref[0] * x


def _prep(t0, x, degp):
    return pl.pallas_call(
        _prep_body,
        grid=(GRID,),
        in_specs=[
            pl.BlockSpec(memory_space=pltpu.SMEM),
            pl.BlockSpec((RB, D), lambda i: (i, 0)),
            pl.BlockSpec((1, RB, DW), lambda i: (0, i, 0)),
            pl.BlockSpec((1, RB, DW), lambda i: (1, i, 0)),
        ],
        out_specs=[
            pl.BlockSpec((RB, 1), lambda i: (i, 0)),
            pl.BlockSpec((RB, D), lambda i: (i, 0)),
            pl.BlockSpec((RB, D), lambda i: (i, 0)),
        ],
        out_shape=[
            jax.ShapeDtypeStruct((N, 1), jnp.float32),
            jax.ShapeDtypeStruct((N, D), jnp.float32),
            jax.ShapeDtypeStruct((N, D), jnp.float32),
        ],
    )(t0, x, degp, degp)


def _combine_body(tk_ref, p0_ref, p1_ref, dis_ref, hid_ref,
                  hid_out_ref, g_out_ref):
    s = p0_ref[0] + p1_ref[0]
    dis = dis_ref[...]
    h = dis * s
    hid_out_ref[...] = hid_ref[...] + tk_ref[0] * h
    g_out_ref[...] = dis * h


def _combine(tk, partial, dis, hid):
    return pl.pallas_call(
        _combine_body,
        grid=(GRID,),
        in_specs=[
            pl.BlockSpec(memory_space=pltpu.SMEM),
            pl.BlockSpec((1, RB, D), lambda i: (0, i, 0)),
            pl.BlockSpec((1, RB, D), lambda i: (1, i, 0)),
            pl.BlockSpec((RB, 1), lambda i: (i, 0)),
            pl.BlockSpec((RB, D), lambda i: (i, 0)),
        ],
        out_specs=[
            pl.BlockSpec((RB, D), lambda i: (i, 0)),
            pl.BlockSpec((RB, D), lambda i: (i, 0)),
        ],
        out_shape=[
            jax.ShapeDtypeStruct((N, D), jnp.float32),
            jax.ShapeDtypeStruct((N, D), jnp.float32),
        ],
    )(tk, partial, partial, dis, hid)


def kernel(x, edge_index, temp):
    loop = jnp.arange(N, dtype=jnp.int32)
    pad_row = jnp.zeros((EP - ET,), jnp.int32)
    pad_col = jnp.full((EP - ET,), N, jnp.int32)
    row = jnp.concatenate([edge_index[0], loop, pad_row]).reshape(NW, CH, CHUNK)
    col = jnp.concatenate([edge_index[1], loop, pad_col]).reshape(NW, CH, CHUNK)

    zeros_d = jnp.zeros((TR, D), jnp.float32)
    zeros_w = jnp.zeros((TR, DW), jnp.float32)
    ones_w = jnp.ones((CHUNK, DW), jnp.float32)

    degp = _deg(col, ones_w, zeros_w)
    dis, g, hidden = _prep(temp[0:1], x, degp)
    for k in range(K):
        partial = _spmv(g, row, col, zeros_d)
        hidden, g = _combine(temp[k + 1:k + 2], partial, dis, hidden)
    return hidden
